# Initial kernel scaffold; baseline (speedup 1.0000x reference)
import jax
import jax.numpy as jnp
from jax.experimental import pallas as pl


def kernel(h, Wq, bq, Wk, bk, Wv, bv, src, dst, non_siblings):
    raise NotImplementedError("write your pallas kernel here")


# speedup vs baseline: 8.4616x; 8.4616x over previous
import functools

import jax
import jax.numpy as jnp
from jax.experimental import pallas as pl
from jax.experimental.pallas import tpu as pltpu

G = 512
NPG = 64
E = 48
H = 8
D = 16
IN = 128
S = 4
GB = 8
NB = G // GB

_HI = jax.lax.Precision.HIGHEST


def _raal_kernel(dst_sref, h_ref, w_ref, b_ref, a_ref, mdst_ref, m01_ref,
                 m23_ref, rt_ref, out_ref, acc_ref):
    hb = h_ref[...]
    qkv = jnp.dot(hb, w_ref[...], preferred_element_type=jnp.float32)
    qkv = qkv + b_ref[0:1, :]
    q = qkv[:, 0:128]
    k = qkv[:, 128:256]
    v = qkv[:, 256:384]

    qsrc = jnp.dot(a_ref[...], q, preferred_element_type=jnp.float32,
                   precision=_HI)
    vsrc = jnp.dot(a_ref[...], v, preferred_element_type=jnp.float32,
                   precision=_HI)

    scores = []
    for hh in range(H):
        qh = qsrc[:, hh * D:(hh + 1) * D]
        kh = k[:, hh * D:(hh + 1) * D]
        c = jax.lax.dot_general(qh, kh, (((1,), (1,)), ((), ())),
                                preferred_element_type=jnp.float32)
        ct = jnp.concatenate(
            [c[g * E:(g + 1) * E, g * NPG:(g + 1) * NPG] for g in range(GB)],
            axis=0)
        cc = jnp.clip(ct, -5.0, 5.0)
        num = jnp.sum(cc * mdst_ref[...], axis=1, keepdims=True)
        d01 = jnp.sum(cc * m01_ref[...], axis=1, keepdims=True)
        d23 = jnp.sum(cc * m23_ref[...], axis=1, keepdims=True)
        den = d01 + d23
        scores.append(num / (den + 1e-6))
    score = jnp.concatenate(scores, axis=1)

    score_w = jnp.dot(score, rt_ref[...], preferred_element_type=jnp.float32,
                      precision=_HI)
    weighted = vsrc * score_w
    packed = jnp.concatenate([weighted, score_w], axis=1)
    p3 = packed.reshape(GB, E, 2 * H * D)

    acc_ref[...] = jnp.zeros_like(acc_ref)
    for e in range(E):
        d = dst_sref[e]
        acc_ref[:, pl.ds(d, 1), :] += p3[:, e:e + 1, :]
    acc = acc_ref[...]
    out = acc[:, :, 0:128] / (acc[:, :, 128:256] + 1e-6)
    out_ref[...] = out.reshape(GB * NPG, H * D)


@jax.jit
def kernel(h, Wq, bq, Wk, bk, Wv, bv, src, dst, non_siblings):
    w = jnp.concatenate([Wq.T, Wk.T, Wv.T], axis=1)
    b = jnp.broadcast_to(jnp.concatenate([bq, bk, bv])[None, :], (8, 3 * H * D))

    cols = jnp.arange(NPG, dtype=jnp.int32)[None, :]
    a1 = (src[:, None] == cols).astype(jnp.float32)
    a_blk = jnp.kron(jnp.eye(GB, dtype=jnp.float32), a1)
    mdst = jnp.tile((dst[:, None] == cols).astype(jnp.float32), (GB, 1))
    ns_oh = [(non_siblings[:, i][:, None] == cols).astype(jnp.float32)
             for i in range(S)]
    m01 = jnp.tile(ns_oh[0] + ns_oh[1], (GB, 1))
    m23 = jnp.tile(ns_oh[2] + ns_oh[3], (GB, 1))
    rt = (jnp.arange(H, dtype=jnp.int32)[:, None]
          == jnp.arange(H * D, dtype=jnp.int32)[None, :] // D).astype(jnp.float32)

    gs = pltpu.PrefetchScalarGridSpec(
        num_scalar_prefetch=1,
        grid=(NB,),
        in_specs=[
            pl.BlockSpec((GB * NPG, IN), lambda i, s: (i, 0)),
            pl.BlockSpec((IN, 3 * H * D), lambda i, s: (0, 0)),
            pl.BlockSpec((8, 3 * H * D), lambda i, s: (0, 0)),
            pl.BlockSpec((GB * E, GB * NPG), lambda i, s: (0, 0)),
            pl.BlockSpec((GB * E, NPG), lambda i, s: (0, 0)),
            pl.BlockSpec((GB * E, NPG), lambda i, s: (0, 0)),
            pl.BlockSpec((GB * E, NPG), lambda i, s: (0, 0)),
            pl.BlockSpec((H, H * D), lambda i, s: (0, 0)),
        ],
        out_specs=pl.BlockSpec((GB * NPG, H * D), lambda i, s: (i, 0)),
        scratch_shapes=[pltpu.VMEM((GB, NPG, 2 * H * D), jnp.float32)],
    )
    out = pl.pallas_call(
        _raal_kernel,
        grid_spec=gs,
        out_shape=jax.ShapeDtypeStruct((G * NPG, H * D), jnp.float32),
    )(dst, h, w, b, a_blk, mdst, m01, m23, rt)
    return out.reshape(G * NPG, H, D)


# speedup vs baseline: 9.8758x; 1.1671x over previous
import functools

import jax
import jax.numpy as jnp
from jax.experimental import pallas as pl
from jax.experimental.pallas import tpu as pltpu

G = 512
NPG = 64
E = 48
H = 8
D = 16
IN = 128
S = 4
GB = 8
NB = G // GB

_HI = jax.lax.Precision.HIGHEST


def _raal_kernel(src_sref, dst_sref, h_ref, w_ref, b_ref, a_ref, mdst_ref,
                 m01_ref, m23_ref, rt_ref, out_ref, acc_ref, accz_ref,
                 vbuf_ref):
    hb = h_ref[...]
    qkv = jnp.dot(hb, w_ref[...], preferred_element_type=jnp.float32)
    qkv = qkv + b_ref[0:1, :]
    q = qkv[:, 0:128]
    k = qkv[:, 128:256]
    v = qkv[:, 256:384]

    qsrc = jnp.dot(a_ref[...], q, preferred_element_type=jnp.float32)

    scores = []
    for hh in range(H):
        qh = qsrc[:, hh * D:(hh + 1) * D]
        kh = k[:, hh * D:(hh + 1) * D]
        c = jax.lax.dot_general(qh, kh, (((1,), (1,)), ((), ())),
                                preferred_element_type=jnp.float32)
        ct = jnp.concatenate(
            [c[g * E:(g + 1) * E, g * NPG:(g + 1) * NPG] for g in range(GB)],
            axis=0)
        cc = jnp.clip(ct, -5.0, 5.0)
        num = jnp.sum(cc * mdst_ref[...], axis=1, keepdims=True)
        d01 = jnp.sum(cc * m01_ref[...], axis=1, keepdims=True)
        d23 = jnp.sum(cc * m23_ref[...], axis=1, keepdims=True)
        den = d01 + d23
        scores.append(num / (den + 1e-6))
    score = jnp.concatenate(scores, axis=1)

    score_w = jnp.dot(score, rt_ref[...], preferred_element_type=jnp.float32,
                      precision=_HI)
    sw3 = score_w.reshape(GB, E, H * D)
    vbuf_ref[...] = v.reshape(GB, NPG, H * D)

    acc_ref[...] = jnp.zeros_like(acc_ref)
    accz_ref[...] = jnp.zeros_like(accz_ref)
    for e in range(E):
        s = src_sref[e]
        d = dst_sref[e]
        sw_e = sw3[:, e:e + 1, :]
        acc_ref[:, pl.ds(d, 1), :] += vbuf_ref[:, pl.ds(s, 1), :] * sw_e
        accz_ref[:, pl.ds(d, 1), :] += sw_e
    out = acc_ref[...] / (accz_ref[...] + 1e-6)
    out_ref[...] = out.reshape(GB * NPG, H * D)


@jax.jit
def kernel(h, Wq, bq, Wk, bk, Wv, bv, src, dst, non_siblings):
    w = jnp.concatenate([Wq.T, Wk.T, Wv.T], axis=1)
    b = jnp.broadcast_to(jnp.concatenate([bq, bk, bv])[None, :], (8, 3 * H * D))

    cols = jnp.arange(NPG, dtype=jnp.int32)[None, :]
    a1 = (src[:, None] == cols).astype(jnp.float32)
    a_blk = jnp.kron(jnp.eye(GB, dtype=jnp.float32), a1)
    mdst = jnp.tile((dst[:, None] == cols).astype(jnp.float32), (GB, 1))
    ns_oh = [(non_siblings[:, i][:, None] == cols).astype(jnp.float32)
             for i in range(S)]
    m01 = jnp.tile(ns_oh[0] + ns_oh[1], (GB, 1))
    m23 = jnp.tile(ns_oh[2] + ns_oh[3], (GB, 1))
    rt = (jnp.arange(H, dtype=jnp.int32)[:, None]
          == jnp.arange(H * D, dtype=jnp.int32)[None, :] // D).astype(jnp.float32)

    gs = pltpu.PrefetchScalarGridSpec(
        num_scalar_prefetch=2,
        grid=(NB,),
        in_specs=[
            pl.BlockSpec((GB * NPG, IN), lambda i, s1, s2: (i, 0)),
            pl.BlockSpec((IN, 3 * H * D), lambda i, s1, s2: (0, 0)),
            pl.BlockSpec((8, 3 * H * D), lambda i, s1, s2: (0, 0)),
            pl.BlockSpec((GB * E, GB * NPG), lambda i, s1, s2: (0, 0)),
            pl.BlockSpec((GB * E, NPG), lambda i, s1, s2: (0, 0)),
            pl.BlockSpec((GB * E, NPG), lambda i, s1, s2: (0, 0)),
            pl.BlockSpec((GB * E, NPG), lambda i, s1, s2: (0, 0)),
            pl.BlockSpec((H, H * D), lambda i, s1, s2: (0, 0)),
        ],
        out_specs=pl.BlockSpec((GB * NPG, H * D), lambda i, s1, s2: (i, 0)),
        scratch_shapes=[pltpu.VMEM((GB, NPG, H * D), jnp.float32),
                        pltpu.VMEM((GB, NPG, H * D), jnp.float32),
                        pltpu.VMEM((GB, NPG, H * D), jnp.float32)],
    )
    out = pl.pallas_call(
        _raal_kernel,
        grid_spec=gs,
        out_shape=jax.ShapeDtypeStruct((G * NPG, H * D), jnp.float32),
    )(src, dst, h, w, b, a_blk, mdst, m01, m23, rt)
    return out.reshape(G * NPG, H, D)


# speedup vs baseline: 13.9993x; 1.4175x over previous
import functools

import jax
import jax.numpy as jnp
from jax.experimental import pallas as pl
from jax.experimental.pallas import tpu as pltpu

G = 512
NPG = 64
E = 48
H = 8
D = 16
IN = 128
S = 4
GB = 8
NB = G // GB

_HI = jax.lax.Precision.HIGHEST


def _raal_kernel(src_sref, dst_sref, h_ref, w_ref, b_ref, a_ref, bm_ref,
                 mdst_ref, m01_ref, m23_ref, rt_ref, out_ref, acc_ref,
                 accz_ref, vbuf_ref):
    hb = h_ref[...]
    qkv = jnp.dot(hb, w_ref[...], preferred_element_type=jnp.float32)
    qkv = qkv + b_ref[0:1, :]
    q = qkv[:, 0:128]
    k = qkv[:, 128:256]
    v = qkv[:, 256:384]

    qsrc = jnp.dot(a_ref[...], q, preferred_element_type=jnp.float32)

    scores = []
    for hh in range(H):
        qh = qsrc[:, hh * D:(hh + 1) * D]
        kh = k[:, hh * D:(hh + 1) * D]
        ct = jnp.concatenate(
            [jax.lax.dot_general(qh[g * E:(g + 1) * E, :],
                                 kh[g * NPG:(g + 1) * NPG, :],
                                 (((1,), (1,)), ((), ())),
                                 preferred_element_type=jnp.float32)
             for g in range(GB)],
            axis=0)
        cc = jnp.clip(ct, -5.0, 5.0)
        num = jnp.sum(cc * mdst_ref[...], axis=1, keepdims=True)
        d01 = jnp.sum(cc * m01_ref[...], axis=1, keepdims=True)
        d23 = jnp.sum(cc * m23_ref[...], axis=1, keepdims=True)
        den = d01 + d23
        scores.append(num / (den + 1e-6))
    score = jnp.concatenate(scores, axis=1)

    score_w = jnp.dot(score, rt_ref[...], preferred_element_type=jnp.float32,
                      precision=_HI)
    sw3 = score_w.reshape(GB, E, H * D)
    vbuf_ref[...] = v.reshape(GB, NPG, H * D)

    acc_ref[...] = jnp.zeros_like(acc_ref)
    accz_ref[...] = jnp.zeros_like(accz_ref)
    for e in range(E):
        s = src_sref[e]
        d = dst_sref[e]
        sw_e = sw3[:, e:e + 1, :]
        acc_ref[:, pl.ds(d, 1), :] += vbuf_ref[:, pl.ds(s, 1), :] * sw_e
        accz_ref[:, pl.ds(d, 1), :] += sw_e
    out = acc_ref[...] / (accz_ref[...] + 1e-6)
    out_ref[...] = out.reshape(GB * NPG, H * D)


@jax.jit
def kernel(h, Wq, bq, Wk, bk, Wv, bv, src, dst, non_siblings):
    w = jnp.concatenate([Wq.T, Wk.T, Wv.T], axis=1)
    b = jnp.broadcast_to(jnp.concatenate([bq, bk, bv])[None, :], (8, 3 * H * D))

    cols = jnp.arange(NPG, dtype=jnp.int32)[None, :]
    a1 = (src[:, None] == cols).astype(jnp.float32)
    a_blk = jnp.kron(jnp.eye(GB, dtype=jnp.float32), a1)
    mdst = jnp.tile((dst[:, None] == cols).astype(jnp.float32), (GB, 1))
    ns_oh = [(non_siblings[:, i][:, None] == cols).astype(jnp.float32)
             for i in range(S)]
    m01 = jnp.tile(ns_oh[0] + ns_oh[1], (GB, 1))
    m23 = jnp.tile(ns_oh[2] + ns_oh[3], (GB, 1))
    rt = (jnp.arange(H, dtype=jnp.int32)[:, None]
          == jnp.arange(H * D, dtype=jnp.int32)[None, :] // D).astype(jnp.float32)
    bm = jnp.kron(jnp.eye(GB, dtype=jnp.float32),
                  jnp.ones((E, NPG), jnp.float32))

    gs = pltpu.PrefetchScalarGridSpec(
        num_scalar_prefetch=2,
        grid=(NB,),
        in_specs=[
            pl.BlockSpec((GB * NPG, IN), lambda i, s1, s2: (i, 0)),
            pl.BlockSpec((IN, 3 * H * D), lambda i, s1, s2: (0, 0)),
            pl.BlockSpec((8, 3 * H * D), lambda i, s1, s2: (0, 0)),
            pl.BlockSpec((GB * E, GB * NPG), lambda i, s1, s2: (0, 0)),
            pl.BlockSpec((GB * E, GB * NPG), lambda i, s1, s2: (0, 0)),
            pl.BlockSpec((GB * E, NPG), lambda i, s1, s2: (0, 0)),
            pl.BlockSpec((GB * E, NPG), lambda i, s1, s2: (0, 0)),
            pl.BlockSpec((GB * E, NPG), lambda i, s1, s2: (0, 0)),
            pl.BlockSpec((H, H * D), lambda i, s1, s2: (0, 0)),
        ],
        out_specs=pl.BlockSpec((GB * NPG, H * D), lambda i, s1, s2: (i, 0)),
        scratch_shapes=[pltpu.VMEM((GB, NPG, H * D), jnp.float32),
                        pltpu.VMEM((GB, NPG, H * D), jnp.float32),
                        pltpu.VMEM((GB, NPG, H * D), jnp.float32)],
    )
    out = pl.pallas_call(
        _raal_kernel,
        grid_spec=gs,
        out_shape=jax.ShapeDtypeStruct((G * NPG, H * D), jnp.float32),
    )(src, dst, h, w, b, a_blk, bm, mdst, m01, m23, rt)
    return out.reshape(G * NPG, H, D)


# speedup vs baseline: 14.9674x; 1.0692x over previous
import functools

import jax
import jax.numpy as jnp
from jax.experimental import pallas as pl
from jax.experimental.pallas import tpu as pltpu

G = 512
NPG = 64
E = 48
H = 8
D = 16
IN = 128
S = 4
GB = 8
NB = G // GB

_HI = jax.lax.Precision.HIGHEST


def _raal_kernel(src_sref, dst_sref, h_ref, w_ref, b_ref, a_ref,
                 mdst_ref, m01_ref, m23_ref, rt_ref, out_ref, acc_ref,
                 vbuf_ref):
    hb = h_ref[...]
    qkv = jnp.dot(hb, w_ref[...], preferred_element_type=jnp.float32)
    qkv = qkv + b_ref[0:1, :]
    q = qkv[:, 0:128]
    k = qkv[:, 128:256]
    v = qkv[:, 256:384]

    qsrc = jnp.dot(a_ref[...], q, preferred_element_type=jnp.float32)

    scores = []
    for hh in range(H):
        qh = qsrc[:, hh * D:(hh + 1) * D]
        kh = k[:, hh * D:(hh + 1) * D]
        ct = jnp.concatenate(
            [jax.lax.dot_general(qh[g * E:(g + 1) * E, :],
                                 kh[g * NPG:(g + 1) * NPG, :],
                                 (((1,), (1,)), ((), ())),
                                 preferred_element_type=jnp.float32)
             for g in range(GB)],
            axis=0)
        cc = jnp.clip(ct, -5.0, 5.0)
        num = jnp.sum(cc * mdst_ref[...], axis=1, keepdims=True)
        d01 = jnp.sum(cc * m01_ref[...], axis=1, keepdims=True)
        d23 = jnp.sum(cc * m23_ref[...], axis=1, keepdims=True)
        den = d01 + d23
        scores.append(num / (den + 1e-6))
    score = jnp.concatenate(scores, axis=1)

    score_w = jnp.dot(score, rt_ref[...], preferred_element_type=jnp.float32,
                      precision=_HI)
    swp = jnp.concatenate([score_w, score_w], axis=1).reshape(GB, E, 2 * H * D)
    v3 = v.reshape(GB, NPG, H * D)
    vbuf_ref[...] = jnp.concatenate([v3, jnp.ones_like(v3)], axis=2)

    acc_ref[...] = jnp.zeros_like(acc_ref)
    for e in range(E):
        s = src_sref[e]
        d = dst_sref[e]
        acc_ref[:, pl.ds(d, 1), :] += (vbuf_ref[:, pl.ds(s, 1), :]
                                       * swp[:, e:e + 1, :])
    acc = acc_ref[...]
    out = acc[:, :, 0:128] / (acc[:, :, 128:256] + 1e-6)
    out_ref[...] = out.reshape(GB * NPG, H * D)


@jax.jit
def kernel(h, Wq, bq, Wk, bk, Wv, bv, src, dst, non_siblings):
    w = jnp.concatenate([Wq.T, Wk.T, Wv.T], axis=1)
    b = jnp.broadcast_to(jnp.concatenate([bq, bk, bv])[None, :], (8, 3 * H * D))

    cols = jnp.arange(NPG, dtype=jnp.int32)[None, :]
    a1 = (src[:, None] == cols).astype(jnp.float32)
    a_blk = jnp.kron(jnp.eye(GB, dtype=jnp.float32), a1)
    mdst = jnp.tile((dst[:, None] == cols).astype(jnp.float32), (GB, 1))
    ns_oh = [(non_siblings[:, i][:, None] == cols).astype(jnp.float32)
             for i in range(S)]
    m01 = jnp.tile(ns_oh[0] + ns_oh[1], (GB, 1))
    m23 = jnp.tile(ns_oh[2] + ns_oh[3], (GB, 1))
    rt = (jnp.arange(H, dtype=jnp.int32)[:, None]
          == jnp.arange(H * D, dtype=jnp.int32)[None, :] // D).astype(jnp.float32)
    gs = pltpu.PrefetchScalarGridSpec(
        num_scalar_prefetch=2,
        grid=(NB,),
        in_specs=[
            pl.BlockSpec((GB * NPG, IN), lambda i, s1, s2: (i, 0)),
            pl.BlockSpec((IN, 3 * H * D), lambda i, s1, s2: (0, 0)),
            pl.BlockSpec((8, 3 * H * D), lambda i, s1, s2: (0, 0)),
            pl.BlockSpec((GB * E, GB * NPG), lambda i, s1, s2: (0, 0)),
            pl.BlockSpec((GB * E, NPG), lambda i, s1, s2: (0, 0)),
            pl.BlockSpec((GB * E, NPG), lambda i, s1, s2: (0, 0)),
            pl.BlockSpec((GB * E, NPG), lambda i, s1, s2: (0, 0)),
            pl.BlockSpec((H, H * D), lambda i, s1, s2: (0, 0)),
        ],
        out_specs=pl.BlockSpec((GB * NPG, H * D), lambda i, s1, s2: (i, 0)),
        scratch_shapes=[pltpu.VMEM((GB, NPG, 2 * H * D), jnp.float32),
                        pltpu.VMEM((GB, NPG, 2 * H * D), jnp.float32)],
    )
    out = pl.pallas_call(
        _raal_kernel,
        grid_spec=gs,
        out_shape=jax.ShapeDtypeStruct((G * NPG, H * D), jnp.float32),
    )(src, dst, h, w, b, a_blk, mdst, m01, m23, rt)
    return out.reshape(G * NPG, H, D)
